# baseline (speedup 1.0000x reference)
import functools

import jax
import jax.numpy as jnp
from jax import lax
from jax.experimental import pallas as pl
from jax.experimental.pallas import tpu as pltpu
from jax.experimental.pallas import tpu_sc as plsc

N = 524288
C = 32
K = 52429
NW = 32
ROWS_W = N // NW
CHUNK = 512
NCH = ROWS_W // CHUNK
UNROLL = 4

MIN32 = -2147483648

ROUNDS_SPEC = ((21, 11, None), (10, 11, 21), (0, 10, 10))


def _make_round(shift: int, bits: int, mask_shift):
    first = mask_shift is None
    nbuckets = 1 << bits
    hsize = nbuckets * C
    mesh = plsc.VectorSubcoreMesh(core_axis_name="c", subcore_axis_name="s")
    out_type = jax.ShapeDtypeStruct((NW, hsize), jnp.int32)
    scratch = [
        pltpu.VMEM((2, CHUNK, C), jnp.float32),
        pltpu.VMEM((hsize,), jnp.int32),
        pltpu.VMEM((C,), jnp.int32),
        pltpu.SemaphoreType.DMA,
        pltpu.SemaphoreType.DMA,
    ]

    def body(*refs):
        if first:
            x_hbm, cnt_hbm, stage, cnt_v, pref_v, sem0, sem1 = refs
            pref_hbm = None
        else:
            x_hbm, pref_hbm, cnt_hbm, stage, cnt_v, pref_v, sem0, sem1 = refs

        wid = lax.axis_index("s") * 2 + lax.axis_index("c")
        row0 = wid * ROWS_W

        zi = jnp.zeros((16,), jnp.int32)

        def zero_body(i, carry):
            cnt_v[pl.ds(i * 16, 16)] = zi
            return carry

        lax.fori_loop(0, hsize // 16, zero_body, 0)

        if not first:
            pltpu.sync_copy(pref_hbm, pref_v)
            pref_lo = pref_v[pl.ds(0, 16)]
            pref_hi = pref_v[pl.ds(16, 16)]
            prefs = (pref_lo, pref_hi)
        else:
            prefs = (None, None)

        iota = lax.iota(jnp.int32, 16)
        cols = (iota, iota + 16)
        ones = jnp.ones((16,), jnp.int32)

        def dma(ch, buf, sem):
            return pltpu.make_async_copy(
                x_hbm.at[pl.ds(row0 + ch * CHUNK, CHUNK)],
                stage.at[buf], sem)

        dma(0, 0, sem0).start()
        sems = (sem0, sem1)
        for ch in range(NCH):
            buf = ch & 1
            dma(ch, buf, sems[buf]).wait()
            if ch + 1 < NCH:
                dma(ch + 1, 1 - buf, sems[1 - buf]).start()

            def chunk_body(j, carry):
                vs, idxs, msks = [], [], []
                for u in range(UNROLL):
                    r = j * UNROLL + u
                    for half in range(2):
                        vs.append(stage[buf, r, pl.ds(16 * half, 16)])
                for i, v in enumerate(vs):
                    half = i & 1
                    y = lax.bitcast_convert_type(v, jnp.int32)
                    m = lax.shift_right_arithmetic(y, 31)
                    key = lax.bitwise_xor(
                        y, lax.bitwise_or(m, jnp.int32(MIN32)))
                    bucket = lax.shift_right_logical(key, shift)
                    if shift + bits < 32:
                        bucket = lax.bitwise_and(
                            bucket, jnp.int32(nbuckets - 1))
                    idxs.append(bucket * 32 + cols[half])
                    if first:
                        msks.append(None)
                    else:
                        keyhi = lax.shift_right_logical(key, mask_shift)
                        msks.append(keyhi == prefs[half])
                for idx, msk in zip(idxs, msks):
                    plsc.addupdate_scatter(cnt_v, [idx], ones, mask=msk)
                return carry

            lax.fori_loop(0, CHUNK // UNROLL, chunk_body, 0)

        pltpu.sync_copy(cnt_v, cnt_hbm.at[wid])

    return pl.kernel(
        body, out_type=out_type, mesh=mesh, scratch_types=scratch,
        compiler_params=pltpu.CompilerParams(
            needs_layout_passes=False, use_tc_tiling_on_sc=False))


_SC_ROUNDS = tuple(
    (spec, _make_round(*spec)) for spec in ROUNDS_SPEC
)

TBLK = 8192


def _tail_body(x_ref, t_ref, s_ref, c_ref):
    i = pl.program_id(0)
    x = x_ref[...]
    t = t_ref[...]
    m = x < t
    s = jnp.sum(jnp.where(m, x, jnp.float32(0)), axis=0, keepdims=True)
    c = jnp.sum(m.astype(jnp.int32), axis=0, keepdims=True)

    @pl.when(i == 0)
    def _():
        s_ref[...] = jnp.zeros_like(s_ref)
        c_ref[...] = jnp.zeros_like(c_ref)

    s_ref[...] += s
    c_ref[...] += c


_tail_pass = pl.pallas_call(
    _tail_body,
    grid=(N // TBLK,),
    in_specs=[
        pl.BlockSpec((TBLK, C), lambda i: (i, 0)),
        pl.BlockSpec((1, C), lambda i: (0, 0)),
    ],
    out_specs=[
        pl.BlockSpec((1, C), lambda i: (0, 0)),
        pl.BlockSpec((1, C), lambda i: (0, 0)),
    ],
    out_shape=(
        jax.ShapeDtypeStruct((1, C), jnp.float32),
        jax.ShapeDtypeStruct((1, C), jnp.int32),
    ),
)


def kernel(input):
    x = input
    k_rem = jnp.full((C,), K, jnp.int32)
    prefix = jnp.zeros((C,), jnp.int32)
    for (shift, bits, mask_shift), fn in _SC_ROUNDS:
        nbuckets = 1 << bits
        if mask_shift is None:
            cnt = fn(x)
        else:
            cnt = fn(x, prefix)
        cntm = cnt.sum(axis=0).reshape(nbuckets, C)
        cum = jnp.cumsum(cntm, axis=0)
        b = jnp.argmax(cum >= k_rem[None, :], axis=0).astype(jnp.int32)
        cnt_below = jnp.take_along_axis(cum - cntm, b[None, :], 0)[0]
        k_rem = k_rem - cnt_below
        prefix = prefix * nbuckets + b
    ybits = jnp.where(prefix < 0, prefix ^ jnp.int32(MIN32), ~prefix)
    tval = lax.bitcast_convert_type(ybits, jnp.float32)
    s, cp = _tail_pass(x, tval[None, :])
    below = (jnp.int32(K) - cp[0]).astype(jnp.float32)
    return -(s[0] + below * tval) / jnp.float32(K)


# speedup vs baseline: 1.2908x; 1.2908x over previous
import functools

import jax
import jax.numpy as jnp
from jax import lax
from jax.experimental import pallas as pl
from jax.experimental.pallas import tpu as pltpu
from jax.experimental.pallas import tpu_sc as plsc

N = 524288
C = 32
K = 52429
NW = 32
ROWS_W = N // NW
CHUNK = 512
NCH = ROWS_W // CHUNK
UNROLL = 4

MIN32 = -2147483648

ROUNDS_SPEC = ((21, 11, None), (10, 11, 21), (0, 10, 10))


def _make_round(shift: int, bits: int, mask_shift):
    first = mask_shift is None
    last = shift == 0
    nbuckets = 1 << bits
    hsize = nbuckets * C
    mesh = plsc.VectorSubcoreMesh(core_axis_name="c", subcore_axis_name="s")
    if last:
        out_type = (
            jax.ShapeDtypeStruct((NW, hsize), jnp.int32),
            jax.ShapeDtypeStruct((NW, hsize), jnp.float32),
            jax.ShapeDtypeStruct((NW, C), jnp.float32),
        )
    else:
        out_type = jax.ShapeDtypeStruct((NW, hsize), jnp.int32)
    scratch = [
        pltpu.VMEM((2, CHUNK, C), jnp.float32),
        pltpu.VMEM((hsize,), jnp.int32),
    ]
    if last:
        scratch += [
            pltpu.VMEM((hsize,), jnp.float32),
            pltpu.VMEM((C,), jnp.float32),
        ]
    scratch += [
        pltpu.VMEM((C,), jnp.int32),
        pltpu.SemaphoreType.DMA,
        pltpu.SemaphoreType.DMA,
    ]

    def body(*refs):
        if first:
            x_hbm, cnt_hbm, stage, cnt_v, pref_v, sem0, sem1 = refs
            pref_hbm = sum_hbm = bel_hbm = sum_v = bel_v = None
        elif last:
            (x_hbm, pref_hbm, cnt_hbm, sum_hbm, bel_hbm, stage, cnt_v,
             sum_v, bel_v, pref_v, sem0, sem1) = refs
        else:
            x_hbm, pref_hbm, cnt_hbm, stage, cnt_v, pref_v, sem0, sem1 = refs
            sum_hbm = bel_hbm = sum_v = bel_v = None

        wid = lax.axis_index("s") * 2 + lax.axis_index("c")
        row0 = wid * ROWS_W

        zi = jnp.zeros((16,), jnp.int32)
        zf = jnp.zeros((16,), jnp.float32)

        def zero_body(i, carry):
            cnt_v[pl.ds(i * 16, 16)] = zi
            if last:
                sum_v[pl.ds(i * 16, 16)] = zf
            return carry

        lax.fori_loop(0, hsize // 16, zero_body, 0)

        if not first:
            pltpu.sync_copy(pref_hbm, pref_v)
            prefs = (pref_v[pl.ds(0, 16)], pref_v[pl.ds(16, 16)])
        else:
            prefs = (None, None)

        iota = lax.iota(jnp.int32, 16)
        cols = (iota, iota + 16)
        ones = jnp.ones((16,), jnp.int32)

        def dma(ch, buf, sem):
            return pltpu.make_async_copy(
                x_hbm.at[pl.ds(row0 + ch * CHUNK, CHUNK)],
                stage.at[buf], sem)

        dma(0, 0, sem0).start()
        sems = (sem0, sem1)
        acc = (zf, zf)
        for ch in range(NCH):
            buf = ch & 1
            dma(ch, buf, sems[buf]).wait()
            if ch + 1 < NCH:
                dma(ch + 1, 1 - buf, sems[1 - buf]).start()

            def chunk_body(j, carry):
                acc_lo, acc_hi = carry
                accs = [acc_lo, acc_hi]
                vs, idxs, msks = [], [], []
                for u in range(UNROLL):
                    r = j * UNROLL + u
                    for half in range(2):
                        vs.append(stage[buf, r, pl.ds(16 * half, 16)])
                for i, v in enumerate(vs):
                    half = i & 1
                    y = lax.bitcast_convert_type(v, jnp.int32)
                    m = lax.shift_right_arithmetic(y, 31)
                    key = lax.bitwise_xor(
                        y, lax.bitwise_or(m, jnp.int32(MIN32)))
                    bucket = lax.shift_right_logical(key, shift)
                    if shift + bits < 32:
                        bucket = lax.bitwise_and(
                            bucket, jnp.int32(nbuckets - 1))
                    idxs.append(bucket * 32 + cols[half])
                    if first:
                        msks.append(None)
                    else:
                        keyhi = lax.shift_right_logical(key, mask_shift)
                        msks.append(keyhi == prefs[half])
                        if last:
                            bel = keyhi < prefs[half]
                            accs[half] = accs[half] + jnp.where(
                                bel, v, jnp.float32(0))
                for i, (idx, msk) in enumerate(zip(idxs, msks)):
                    plsc.addupdate_scatter(cnt_v, [idx], ones, mask=msk)
                    if last:
                        plsc.addupdate_scatter(sum_v, [idx], vs[i], mask=msk)
                return (accs[0], accs[1])

            acc = lax.fori_loop(0, CHUNK // UNROLL, chunk_body, acc)

        pltpu.sync_copy(cnt_v, cnt_hbm.at[wid])
        if last:
            bel_v[pl.ds(0, 16)] = acc[0]
            bel_v[pl.ds(16, 16)] = acc[1]
            pltpu.sync_copy(sum_v, sum_hbm.at[wid])
            pltpu.sync_copy(bel_v, bel_hbm.at[wid])

    return pl.kernel(
        body, out_type=out_type, mesh=mesh, scratch_types=scratch,
        compiler_params=pltpu.CompilerParams(
            needs_layout_passes=False, use_tc_tiling_on_sc=False))


_SC_ROUNDS = tuple((spec, _make_round(*spec)) for spec in ROUNDS_SPEC)


def kernel(input):
    x = input
    k_rem = jnp.full((C,), K, jnp.int32)
    prefix = jnp.zeros((C,), jnp.int32)
    sum_below = None
    for (shift, bits, mask_shift), fn in _SC_ROUNDS:
        nbuckets = 1 << bits
        if mask_shift is None:
            cnt = fn(x)
        elif shift != 0:
            cnt = fn(x, prefix)
        else:
            cnt, sm, bel = fn(x, prefix)
        cntm = cnt.sum(axis=0).reshape(nbuckets, C)
        cum = jnp.cumsum(cntm, axis=0)
        b = jnp.argmax(cum >= k_rem[None, :], axis=0).astype(jnp.int32)
        cnt_below = jnp.take_along_axis(cum - cntm, b[None, :], 0)[0]
        if shift == 0:
            smm = sm.sum(axis=0).reshape(nbuckets, C)
            in_class_below = jnp.take_along_axis(
                jnp.cumsum(smm, axis=0) - smm, b[None, :], 0)[0]
            sum_below = bel.sum(axis=0) + in_class_below
        k_rem = k_rem - cnt_below
        prefix = prefix * nbuckets + b
    ybits = jnp.where(prefix < 0, prefix ^ jnp.int32(MIN32), ~prefix)
    tval = lax.bitcast_convert_type(ybits, jnp.float32)
    return -(sum_below + k_rem.astype(jnp.float32) * tval) / jnp.float32(K)


# speedup vs baseline: 1.7196x; 1.3321x over previous
import functools

import jax
import jax.numpy as jnp
from jax import lax
from jax.experimental import pallas as pl
from jax.experimental.pallas import tpu as pltpu
from jax.experimental.pallas import tpu_sc as plsc

N = 524288
C = 32
K = 52429
NW = 32
CHUNK = 16384
NCH = N // CHUNK
UNROLL = 8

MIN32 = -2147483648

ROUNDS_SPEC = ((21, 11, None), (10, 11, 21), (0, 10, 10))


def _make_round(shift: int, bits: int, mask_shift):
    first = mask_shift is None
    last = shift == 0
    nbuckets = 1 << bits
    hsize = nbuckets * 16
    mesh = plsc.VectorSubcoreMesh(core_axis_name="c", subcore_axis_name="s")
    if last:
        out_type = (
            jax.ShapeDtypeStruct((NW, hsize), jnp.int32),
            jax.ShapeDtypeStruct((NW, hsize), jnp.float32),
            jax.ShapeDtypeStruct((NW, 16), jnp.float32),
        )
    else:
        out_type = jax.ShapeDtypeStruct((NW, hsize), jnp.int32)
    scratch = [
        pltpu.VMEM((2, CHUNK), jnp.float32),
        pltpu.VMEM((hsize,), jnp.int32),
    ]
    if last:
        scratch += [
            pltpu.VMEM((hsize,), jnp.float32),
            pltpu.VMEM((16,), jnp.float32),
        ]
    scratch += [
        pltpu.VMEM((C,), jnp.int32),
        pltpu.SemaphoreType.DMA,
        pltpu.SemaphoreType.DMA,
    ]

    def body(*refs):
        if first:
            xt_hbm, cnt_hbm, stage, cnt_v, pref_v, sem0, sem1 = refs
            pref_hbm = sum_hbm = bel_hbm = sum_v = bel_v = None
        elif last:
            (xt_hbm, pref_hbm, cnt_hbm, sum_hbm, bel_hbm, stage, cnt_v,
             sum_v, bel_v, pref_v, sem0, sem1) = refs
        else:
            (xt_hbm, pref_hbm, cnt_hbm, stage, cnt_v, pref_v,
             sem0, sem1) = refs
            sum_hbm = bel_hbm = sum_v = bel_v = None

        wid = lax.axis_index("s") * 2 + lax.axis_index("c")

        zi = jnp.zeros((16,), jnp.int32)
        zf = jnp.zeros((16,), jnp.float32)

        def zero_body(i, carry):
            cnt_v[pl.ds(i * 16, 16)] = zi
            if last:
                sum_v[pl.ds(i * 16, 16)] = zf
            return carry

        lax.fori_loop(0, hsize // 16, zero_body, 0)

        iota = lax.iota(jnp.int32, 16)
        ones = jnp.ones((16,), jnp.int32)

        if not first:
            pltpu.sync_copy(pref_hbm, pref_v)
            widv = lax.broadcast_in_dim(wid, (16,), ())
            pref = plsc.load_gather(pref_v, [widv])
        else:
            pref = None

        def dma(ch, buf, sem):
            return pltpu.make_async_copy(
                xt_hbm.at[wid, pl.ds(ch * CHUNK, CHUNK)],
                stage.at[buf], sem)

        dma(0, 0, sem0).start()
        sems = (sem0, sem1)
        acc = zf
        for ch in range(NCH):
            buf = ch & 1
            dma(ch, buf, sems[buf]).wait()
            if ch + 1 < NCH:
                dma(ch + 1, 1 - buf, sems[1 - buf]).start()

            def chunk_body(j, carry):
                acc_l = carry
                vs, idxs, msks = [], [], []
                for u in range(UNROLL):
                    vs.append(stage[buf, pl.ds((j * UNROLL + u) * 16, 16)])
                for v in vs:
                    y = lax.bitcast_convert_type(v, jnp.int32)
                    m = lax.shift_right_arithmetic(y, 31)
                    key = lax.bitwise_xor(
                        y, lax.bitwise_or(m, jnp.int32(MIN32)))
                    bucket = lax.shift_right_logical(key, shift)
                    if shift + bits < 32:
                        bucket = lax.bitwise_and(
                            bucket, jnp.int32(nbuckets - 1))
                    idxs.append(bucket * 16 + iota)
                    if first:
                        msks.append(None)
                    else:
                        keyhi = lax.shift_right_logical(key, mask_shift)
                        msks.append(keyhi == pref)
                        if last:
                            bel = keyhi < pref
                            acc_l = acc_l + jnp.where(bel, v, jnp.float32(0))
                for i, (idx, msk) in enumerate(zip(idxs, msks)):
                    plsc.addupdate_scatter(cnt_v, [idx], ones, mask=msk)
                    if last:
                        plsc.addupdate_scatter(sum_v, [idx], vs[i], mask=msk)
                return acc_l

            acc = lax.fori_loop(0, CHUNK // 16 // UNROLL, chunk_body, acc)

        pltpu.sync_copy(cnt_v, cnt_hbm.at[wid])
        if last:
            bel_v[pl.ds(0, 16)] = acc
            pltpu.sync_copy(sum_v, sum_hbm.at[wid])
            pltpu.sync_copy(bel_v, bel_hbm.at[wid])

    return pl.kernel(
        body, out_type=out_type, mesh=mesh, scratch_types=scratch,
        compiler_params=pltpu.CompilerParams(
            needs_layout_passes=False, use_tc_tiling_on_sc=False))


_SC_ROUNDS = tuple((spec, _make_round(*spec)) for spec in ROUNDS_SPEC)


def kernel(input):
    xt = input.T
    k_rem = jnp.full((C,), K, jnp.int32)
    prefix = jnp.zeros((C,), jnp.int32)
    sum_below = None
    for (shift, bits, mask_shift), fn in _SC_ROUNDS:
        nbuckets = 1 << bits
        if mask_shift is None:
            cnt = fn(xt)
        elif shift != 0:
            cnt = fn(xt, prefix)
        else:
            cnt, sm, bel = fn(xt, prefix)
        cntm = cnt.reshape(C, nbuckets, 16).sum(axis=-1).T
        cum = jnp.cumsum(cntm, axis=0)
        b = jnp.argmax(cum >= k_rem[None, :], axis=0).astype(jnp.int32)
        cnt_below = jnp.take_along_axis(cum - cntm, b[None, :], 0)[0]
        if shift == 0:
            smm = sm.reshape(C, nbuckets, 16).sum(axis=-1).T
            in_class_below = jnp.take_along_axis(
                jnp.cumsum(smm, axis=0) - smm, b[None, :], 0)[0]
            sum_below = bel.sum(axis=-1) + in_class_below
        k_rem = k_rem - cnt_below
        prefix = prefix * nbuckets + b
    ybits = jnp.where(prefix < 0, prefix ^ jnp.int32(MIN32), ~prefix)
    tval = lax.bitcast_convert_type(ybits, jnp.float32)
    return -(sum_below + k_rem.astype(jnp.float32) * tval) / jnp.float32(K)


# speedup vs baseline: 1.8423x; 1.0714x over previous
import functools

import jax
import jax.numpy as jnp
from jax import lax
from jax.experimental import pallas as pl
from jax.experimental.pallas import tpu as pltpu
from jax.experimental.pallas import tpu_sc as plsc

N = 524288
C = 32
K = 52429
NW = 32
CHUNK = 16384
NCH = N // CHUNK
UNROLL = 8

MIN32 = -2147483648

ROUNDS_SPEC = ((21, 11, None), (10, 11, 21), (0, 10, 10))


def _make_round(shift: int, bits: int, mask_shift):
    first = mask_shift is None
    last = shift == 0
    nbuckets = 1 << bits
    hsize = nbuckets * 16
    mesh = plsc.VectorSubcoreMesh(core_axis_name="c", subcore_axis_name="s")
    if last:
        out_type = (
            jax.ShapeDtypeStruct((NW, nbuckets), jnp.int32),
            jax.ShapeDtypeStruct((NW, nbuckets), jnp.float32),
            jax.ShapeDtypeStruct((NW, 16), jnp.float32),
        )
    else:
        out_type = jax.ShapeDtypeStruct((NW, nbuckets), jnp.int32)
    scratch = [
        pltpu.VMEM((2, CHUNK), jnp.float32),
        pltpu.VMEM((hsize,), jnp.int32),
        pltpu.VMEM((nbuckets,), jnp.int32),
    ]
    if last:
        scratch += [
            pltpu.VMEM((hsize,), jnp.float32),
            pltpu.VMEM((nbuckets,), jnp.float32),
            pltpu.VMEM((16,), jnp.float32),
        ]
    scratch += [
        pltpu.VMEM((C,), jnp.int32),
        pltpu.SemaphoreType.DMA,
        pltpu.SemaphoreType.DMA,
    ]

    def body(*refs):
        if first:
            xt_hbm, cnt_hbm, stage, cnt_v, cntf_v, pref_v, sem0, sem1 = refs
            pref_hbm = sum_hbm = bel_hbm = sum_v = smf_v = bel_v = None
        elif last:
            (xt_hbm, pref_hbm, cnt_hbm, sum_hbm, bel_hbm, stage, cnt_v,
             cntf_v, sum_v, smf_v, bel_v, pref_v, sem0, sem1) = refs
        else:
            (xt_hbm, pref_hbm, cnt_hbm, stage, cnt_v, cntf_v, pref_v,
             sem0, sem1) = refs
            sum_hbm = bel_hbm = sum_v = smf_v = bel_v = None

        wid = lax.axis_index("s") * 2 + lax.axis_index("c")

        zi = jnp.zeros((16,), jnp.int32)
        zf = jnp.zeros((16,), jnp.float32)

        def zero_body(i, carry):
            cnt_v[pl.ds(i * 16, 16)] = zi
            if last:
                sum_v[pl.ds(i * 16, 16)] = zf
            return carry

        lax.fori_loop(0, hsize // 16, zero_body, 0)

        iota = lax.iota(jnp.int32, 16)
        ones = jnp.ones((16,), jnp.int32)

        if not first:
            pltpu.sync_copy(pref_hbm, pref_v)
            widv = lax.broadcast_in_dim(wid, (16,), ())
            pref = plsc.load_gather(pref_v, [widv])
        else:
            pref = None

        def dma(ch, buf, sem):
            return pltpu.make_async_copy(
                xt_hbm.at[wid, pl.ds(ch * CHUNK, CHUNK)],
                stage.at[buf], sem)

        dma(0, 0, sem0).start()
        sems = (sem0, sem1)
        acc = zf
        for ch in range(NCH):
            buf = ch & 1
            dma(ch, buf, sems[buf]).wait()
            if ch + 1 < NCH:
                dma(ch + 1, 1 - buf, sems[1 - buf]).start()

            def chunk_body(j, carry):
                acc_l = carry
                vs, idxs, msks = [], [], []
                for u in range(UNROLL):
                    vs.append(stage[buf, pl.ds((j * UNROLL + u) * 16, 16)])
                for v in vs:
                    y = lax.bitcast_convert_type(v, jnp.int32)
                    m = lax.shift_right_arithmetic(y, 31)
                    key = lax.bitwise_xor(
                        y, lax.bitwise_or(m, jnp.int32(MIN32)))
                    bucket = lax.shift_right_logical(key, shift)
                    if shift + bits < 32:
                        bucket = lax.bitwise_and(
                            bucket, jnp.int32(nbuckets - 1))
                    idxs.append(bucket * 16 + iota)
                    if first:
                        msks.append(None)
                    else:
                        keyhi = lax.shift_right_logical(key, mask_shift)
                        msks.append(keyhi == pref)
                        if last:
                            bel = keyhi < pref
                            acc_l = acc_l + jnp.where(bel, v, jnp.float32(0))
                for i, (idx, msk) in enumerate(zip(idxs, msks)):
                    plsc.addupdate_scatter(cnt_v, [idx], ones, mask=msk)
                    if last:
                        plsc.addupdate_scatter(sum_v, [idx], vs[i], mask=msk)
                return acc_l

            acc = lax.fori_loop(0, CHUNK // 16 // UNROLL, chunk_body, acc)

        iota16 = iota * 16

        def fold_body(i, carry):
            base = iota16 + i * 256
            ci = plsc.load_gather(cnt_v, [base])
            if last:
                si = plsc.load_gather(sum_v, [base])
            for l in range(1, 16):
                ci = ci + plsc.load_gather(cnt_v, [base + l])
                if last:
                    si = si + plsc.load_gather(sum_v, [base + l])
            cntf_v[pl.ds(i * 16, 16)] = ci
            if last:
                smf_v[pl.ds(i * 16, 16)] = si
            return carry

        lax.fori_loop(0, nbuckets // 16, fold_body, 0)

        pltpu.sync_copy(cntf_v, cnt_hbm.at[wid])
        if last:
            bel_v[pl.ds(0, 16)] = acc
            pltpu.sync_copy(smf_v, sum_hbm.at[wid])
            pltpu.sync_copy(bel_v, bel_hbm.at[wid])

    return pl.kernel(
        body, out_type=out_type, mesh=mesh, scratch_types=scratch,
        compiler_params=pltpu.CompilerParams(
            needs_layout_passes=False, use_tc_tiling_on_sc=False))


_SC_ROUNDS = tuple((spec, _make_round(*spec)) for spec in ROUNDS_SPEC)


def kernel(input):
    xt = input.T
    k_rem = jnp.full((C,), K, jnp.int32)
    prefix = jnp.zeros((C,), jnp.int32)
    sum_below = None
    for (shift, bits, mask_shift), fn in _SC_ROUNDS:
        nbuckets = 1 << bits
        if mask_shift is None:
            cnt = fn(xt)
        elif shift != 0:
            cnt = fn(xt, prefix)
        else:
            cnt, sm, bel = fn(xt, prefix)
        cntm = cnt.T
        cum = jnp.cumsum(cntm, axis=0)
        b = jnp.argmax(cum >= k_rem[None, :], axis=0).astype(jnp.int32)
        cnt_below = jnp.take_along_axis(cum - cntm, b[None, :], 0)[0]
        if shift == 0:
            smm = sm.T
            in_class_below = jnp.take_along_axis(
                jnp.cumsum(smm, axis=0) - smm, b[None, :], 0)[0]
            sum_below = bel.sum(axis=-1) + in_class_below
        k_rem = k_rem - cnt_below
        prefix = prefix * nbuckets + b
    ybits = jnp.where(prefix < 0, prefix ^ jnp.int32(MIN32), ~prefix)
    tval = lax.bitcast_convert_type(ybits, jnp.float32)
    return -(sum_below + k_rem.astype(jnp.float32) * tval) / jnp.float32(K)


# speedup vs baseline: 2.0927x; 1.1359x over previous
import functools

import jax
import jax.numpy as jnp
from jax import lax
from jax.experimental import pallas as pl
from jax.experimental.pallas import tpu as pltpu
from jax.experimental.pallas import tpu_sc as plsc

N = 524288
C = 32
K = 52429
NW = 32
CHUNK = 16384
NCH = N // CHUNK
UNROLL = 8

MIN32 = -2147483648

ROUNDS_SPEC = ((21, 11, None), (10, 11, 21), (0, 10, 10))


def _make_round(shift: int, bits: int, mask_shift):
    first = mask_shift is None
    last = shift == 0
    nbuckets = 1 << bits
    hsize = nbuckets * 16
    mesh = plsc.VectorSubcoreMesh(core_axis_name="c", subcore_axis_name="s")
    if last:
        out_type = (
            jax.ShapeDtypeStruct((NW, nbuckets), jnp.int32),
            jax.ShapeDtypeStruct((NW, nbuckets), jnp.int32),
            jax.ShapeDtypeStruct((NW, nbuckets), jnp.float32),
            jax.ShapeDtypeStruct((NW, 16), jnp.float32),
        )
    else:
        out_type = (
            jax.ShapeDtypeStruct((NW, nbuckets), jnp.int32),
            jax.ShapeDtypeStruct((NW, nbuckets), jnp.int32),
        )
    scratch = [
        pltpu.VMEM((2, CHUNK), jnp.float32),
        pltpu.VMEM((hsize,), jnp.int32),
        pltpu.VMEM((nbuckets,), jnp.int32),
        pltpu.VMEM((nbuckets,), jnp.int32),
    ]
    if last:
        scratch += [
            pltpu.VMEM((hsize,), jnp.float32),
            pltpu.VMEM((nbuckets,), jnp.float32),
            pltpu.VMEM((16,), jnp.float32),
        ]
    scratch += [
        pltpu.VMEM((C,), jnp.int32),
        pltpu.SemaphoreType.DMA,
        pltpu.SemaphoreType.DMA,
    ]

    def body(*refs):
        if first:
            (xt_hbm, cnt_hbm, cum_hbm, stage, cnt_v, cntf_v, cumf_v,
             pref_v, sem0, sem1) = refs
            pref_hbm = sum_hbm = bel_hbm = sum_v = smf_v = bel_v = None
        elif last:
            (xt_hbm, pref_hbm, cnt_hbm, cum_hbm, sum_hbm, bel_hbm, stage,
             cnt_v, cntf_v, cumf_v, sum_v, smf_v, bel_v, pref_v,
             sem0, sem1) = refs
        else:
            (xt_hbm, pref_hbm, cnt_hbm, cum_hbm, stage, cnt_v, cntf_v,
             cumf_v, pref_v, sem0, sem1) = refs
            sum_hbm = bel_hbm = sum_v = smf_v = bel_v = None

        wid = lax.axis_index("s") * 2 + lax.axis_index("c")

        zi = jnp.zeros((16,), jnp.int32)
        zf = jnp.zeros((16,), jnp.float32)

        def zero_body(i, carry):
            cnt_v[pl.ds(i * 16, 16)] = zi
            if last:
                sum_v[pl.ds(i * 16, 16)] = zf
            return carry

        lax.fori_loop(0, hsize // 16, zero_body, 0)

        iota = lax.iota(jnp.int32, 16)
        ones = jnp.ones((16,), jnp.int32)

        if not first:
            pltpu.sync_copy(pref_hbm, pref_v)
            widv = lax.broadcast_in_dim(wid, (16,), ())
            pref = plsc.load_gather(pref_v, [widv])
        else:
            pref = None

        def dma(ch, buf, sem):
            return pltpu.make_async_copy(
                xt_hbm.at[wid, pl.ds(ch * CHUNK, CHUNK)],
                stage.at[buf], sem)

        dma(0, 0, sem0).start()
        sems = (sem0, sem1)
        acc = zf
        for ch in range(NCH):
            buf = ch & 1
            dma(ch, buf, sems[buf]).wait()
            if ch + 1 < NCH:
                dma(ch + 1, 1 - buf, sems[1 - buf]).start()

            def chunk_body(j, carry):
                acc_l = carry
                vs, idxs, msks = [], [], []
                for u in range(UNROLL):
                    vs.append(stage[buf, pl.ds((j * UNROLL + u) * 16, 16)])
                for v in vs:
                    y = lax.bitcast_convert_type(v, jnp.int32)
                    m = lax.shift_right_arithmetic(y, 31)
                    key = lax.bitwise_xor(
                        y, lax.bitwise_or(m, jnp.int32(MIN32)))
                    bucket = lax.shift_right_logical(key, shift)
                    if shift + bits < 32:
                        bucket = lax.bitwise_and(
                            bucket, jnp.int32(nbuckets - 1))
                    idxs.append(bucket * 16 + iota)
                    if first:
                        msks.append(None)
                    else:
                        keyhi = lax.shift_right_logical(key, mask_shift)
                        msks.append(keyhi == pref)
                        if last:
                            bel = keyhi < pref
                            acc_l = acc_l + jnp.where(bel, v, jnp.float32(0))
                for i, (idx, msk) in enumerate(zip(idxs, msks)):
                    plsc.addupdate_scatter(cnt_v, [idx], ones, mask=msk)
                    if last:
                        plsc.addupdate_scatter(sum_v, [idx], vs[i], mask=msk)
                return acc_l

            acc = lax.fori_loop(0, CHUNK // 16 // UNROLL, chunk_body, acc)

        iota16 = iota * 16

        def fold_body(i, carry):
            ccar, scar = carry
            base = iota16 + i * 256
            ci = plsc.load_gather(cnt_v, [base])
            if last:
                si = plsc.load_gather(sum_v, [base])
            for l in range(1, 16):
                ci = ci + plsc.load_gather(cnt_v, [base + l])
                if last:
                    si = si + plsc.load_gather(sum_v, [base + l])
            cntf_v[pl.ds(i * 16, 16)] = ci
            cumf_v[pl.ds(i * 16, 16)] = (
                plsc.cumsum(ci) + lax.broadcast_in_dim(ccar, (16,), ()))
            ccar = ccar + jnp.sum(ci)
            if last:
                smf_v[pl.ds(i * 16, 16)] = (
                    plsc.cumsum(si) - si + lax.broadcast_in_dim(scar, (16,), ()))
                scar = scar + jnp.sum(si)
            return (ccar, scar)

        lax.fori_loop(0, nbuckets // 16, fold_body,
                      (jnp.int32(0), jnp.float32(0)))

        pltpu.sync_copy(cntf_v, cnt_hbm.at[wid])
        pltpu.sync_copy(cumf_v, cum_hbm.at[wid])
        if last:
            bel_v[pl.ds(0, 16)] = acc
            pltpu.sync_copy(smf_v, sum_hbm.at[wid])
            pltpu.sync_copy(bel_v, bel_hbm.at[wid])

    return pl.kernel(
        body, out_type=out_type, mesh=mesh, scratch_types=scratch,
        compiler_params=pltpu.CompilerParams(
            needs_layout_passes=False, use_tc_tiling_on_sc=False))


_SC_ROUNDS = tuple((spec, _make_round(*spec)) for spec in ROUNDS_SPEC)


def kernel(input):
    xt = input.T
    k_rem = jnp.full((C,), K, jnp.int32)
    prefix = jnp.zeros((C,), jnp.int32)
    sum_below = None
    for (shift, bits, mask_shift), fn in _SC_ROUNDS:
        nbuckets = 1 << bits
        if mask_shift is None:
            cnt, cum = fn(xt)
        elif shift != 0:
            cnt, cum = fn(xt, prefix)
        else:
            cnt, cum, smex, bel = fn(xt, prefix)
        cntm = cnt.T
        cum = cum.T
        b = jnp.argmax(cum >= k_rem[None, :], axis=0).astype(jnp.int32)
        cnt_below = jnp.take_along_axis(cum - cntm, b[None, :], 0)[0]
        if shift == 0:
            in_class_below = jnp.take_along_axis(smex.T, b[None, :], 0)[0]
            sum_below = bel.sum(axis=-1) + in_class_below
        k_rem = k_rem - cnt_below
        prefix = prefix * nbuckets + b
    ybits = jnp.where(prefix < 0, prefix ^ jnp.int32(MIN32), ~prefix)
    tval = lax.bitcast_convert_type(ybits, jnp.float32)
    return -(sum_below + k_rem.astype(jnp.float32) * tval) / jnp.float32(K)
